# baseline (speedup 1.0000x reference)
import functools

import jax
import jax.numpy as jnp
from jax import lax
from jax.experimental import pallas as pl
from jax.experimental.pallas import tpu as pltpu
from jax.experimental.pallas import tpu_sc as plsc

_NUM_CORES = 2
_NUM_SUBCORES = 16
_NUM_WORKERS = _NUM_CORES * _NUM_SUBCORES


_HBM_SPLIT = 2432


def _remap_body(b_per_w, vocab, keys_hbm, table_hbm, out_hbm, idx_v, rows_v,
                tab_sh, sem_h, sem_s):
    s = lax.axis_index("s")
    wid = s * _NUM_CORES + lax.axis_index("c")
    base = wid * b_per_w
    n_s = b_per_w - _HBM_SPLIT
    pltpu.sync_copy(keys_hbm.at[pl.ds(base, b_per_w)], idx_v)
    hcopy = pltpu.async_copy(
        table_hbm.at[idx_v.at[pl.ds(0, _HBM_SPLIT)]],
        rows_v.at[pl.ds(0, _HBM_SPLIT)], sem_h)
    del vocab

    @pl.when(s == 0)
    def _stage_table():
        pltpu.sync_copy(table_hbm, tab_sh)

    plsc.subcore_barrier()
    scopy = pltpu.async_copy(
        tab_sh.at[idx_v.at[pl.ds(_HBM_SPLIT, n_s)]],
        rows_v.at[pl.ds(_HBM_SPLIT, n_s)], sem_s)
    hcopy.wait()
    scopy.wait()
    pltpu.sync_copy(rows_v, out_hbm.at[pl.ds(base, b_per_w)])


def kernel(offsets, keys, values, id2index):
    total = keys.shape[0]
    b_per_w = total // _NUM_WORKERS
    mesh = plsc.VectorSubcoreMesh(core_axis_name="c", subcore_axis_name="s")
    remap = pl.kernel(
        functools.partial(_remap_body, b_per_w, id2index.shape[0]),
        mesh=mesh,
        out_type=jax.ShapeDtypeStruct((total,), jnp.int32),
        scratch_types=[
            pltpu.VMEM((b_per_w,), jnp.int32),
            pltpu.VMEM((b_per_w,), jnp.int32),
            pltpu.VMEM_SHARED((id2index.shape[0],), jnp.int32),
            pltpu.SemaphoreType.DMA,
            pltpu.SemaphoreType.DMA,
        ],
    )
    idx_keys = remap(keys, id2index)
    return (offsets, idx_keys, values)


# speedup vs baseline: 1.0303x; 1.0303x over previous
import functools

import jax
import jax.numpy as jnp
from jax import lax
from jax.experimental import pallas as pl
from jax.experimental.pallas import tpu as pltpu
from jax.experimental.pallas import tpu_sc as plsc

_NUM_CORES = 2
_NUM_SUBCORES = 16
_NUM_WORKERS = _NUM_CORES * _NUM_SUBCORES


_HBM_SPLIT = 1280


def _remap_body(b_per_w, vocab, keys_hbm, table_hbm, out_hbm, idx_v, rows_v,
                tab_sh, sem_h, sem_s):
    s = lax.axis_index("s")
    wid = s * _NUM_CORES + lax.axis_index("c")
    base = wid * b_per_w
    n_s = b_per_w - _HBM_SPLIT
    pltpu.sync_copy(keys_hbm.at[pl.ds(base, b_per_w)], idx_v)
    hcopy = pltpu.async_copy(
        table_hbm.at[idx_v.at[pl.ds(0, _HBM_SPLIT)]],
        rows_v.at[pl.ds(0, _HBM_SPLIT)], sem_h)
    del vocab

    @pl.when(s == 0)
    def _stage_table():
        pltpu.sync_copy(table_hbm, tab_sh)

    plsc.subcore_barrier()
    scopy = pltpu.async_copy(
        tab_sh.at[idx_v.at[pl.ds(_HBM_SPLIT, n_s)]],
        rows_v.at[pl.ds(_HBM_SPLIT, n_s)], sem_s)
    hcopy.wait()
    scopy.wait()
    pltpu.sync_copy(rows_v, out_hbm.at[pl.ds(base, b_per_w)])


def kernel(offsets, keys, values, id2index):
    total = keys.shape[0]
    b_per_w = total // _NUM_WORKERS
    mesh = plsc.VectorSubcoreMesh(core_axis_name="c", subcore_axis_name="s")
    remap = pl.kernel(
        functools.partial(_remap_body, b_per_w, id2index.shape[0]),
        mesh=mesh,
        out_type=jax.ShapeDtypeStruct((total,), jnp.int32),
        scratch_types=[
            pltpu.VMEM((b_per_w,), jnp.int32),
            pltpu.VMEM((b_per_w,), jnp.int32),
            pltpu.VMEM_SHARED((id2index.shape[0],), jnp.int32),
            pltpu.SemaphoreType.DMA,
            pltpu.SemaphoreType.DMA,
        ],
    )
    idx_keys = remap(keys, id2index)
    return (offsets, idx_keys, values)


# speedup vs baseline: 1.1061x; 1.0736x over previous
import functools

import jax
import jax.numpy as jnp
from jax import lax
from jax.experimental import pallas as pl
from jax.experimental.pallas import tpu as pltpu
from jax.experimental.pallas import tpu_sc as plsc

_NUM_CORES = 2
_NUM_SUBCORES = 16
_NUM_WORKERS = _NUM_CORES * _NUM_SUBCORES


_HBM_SPLIT = 1280


def _remap_body(b_per_w, vocab, keys_hbm, table_hbm, out_hbm, idx_v, rows_v,
                tab_sh, sem_h, sem_s):
    s = lax.axis_index("s")
    wid = s * _NUM_CORES + lax.axis_index("c")
    base = wid * b_per_w
    n_s = b_per_w - _HBM_SPLIT
    pltpu.sync_copy(keys_hbm.at[pl.ds(base, b_per_w)], idx_v)
    hcopy = pltpu.async_copy(
        table_hbm.at[idx_v.at[pl.ds(0, _HBM_SPLIT)]],
        rows_v.at[pl.ds(0, _HBM_SPLIT)], sem_h)
    del vocab

    @pl.when(s == 0)
    def _stage_table():
        pltpu.sync_copy(table_hbm, tab_sh)

    plsc.subcore_barrier()
    scopy = pltpu.async_copy(
        tab_sh.at[idx_v.at[pl.ds(_HBM_SPLIT, n_s)]],
        rows_v.at[pl.ds(_HBM_SPLIT, n_s)], sem_s)
    hcopy.wait()
    scopy.wait()
    pltpu.sync_copy(rows_v, out_hbm.at[pl.ds(base, b_per_w)])


def kernel(offsets, keys, values, id2index):
    total = keys.shape[0]
    batch = offsets.shape[0]
    b_per_w = total // _NUM_WORKERS
    mesh = plsc.VectorSubcoreMesh(core_axis_name="c", subcore_axis_name="s")
    remap = pl.kernel(
        functools.partial(_remap_body, b_per_w, id2index.shape[0]),
        mesh=mesh,
        out_type=jax.ShapeDtypeStruct((total,), jnp.int32),
        scratch_types=[
            pltpu.VMEM((b_per_w,), jnp.int32),
            pltpu.VMEM((b_per_w,), jnp.int32),
            pltpu.VMEM_SHARED((id2index.shape[0],), jnp.int32),
            pltpu.SemaphoreType.DMA,
            pltpu.SemaphoreType.DMA,
        ],
    )
    idx_keys = remap(keys, id2index)

    def _tc_copy_body(vals_ref, offs_ref, vals_out, offs_out):
        vals_out[...] = vals_ref[...]
        offs_out[...] = offs_ref[...]

    vals2d = values.reshape(total // 128, 128)
    offs2d = offsets.reshape(batch // 128, 128)
    vals_out, offs_out = pl.pallas_call(
        _tc_copy_body,
        out_shape=(
            jax.ShapeDtypeStruct(vals2d.shape, values.dtype),
            jax.ShapeDtypeStruct(offs2d.shape, offsets.dtype),
        ),
    )(vals2d, offs2d)
    return (offs_out.reshape(batch), idx_keys, vals_out.reshape(total))


# speedup vs baseline: 1.1071x; 1.0009x over previous
import functools

import jax
import jax.numpy as jnp
from jax import lax
from jax.experimental import pallas as pl
from jax.experimental.pallas import tpu as pltpu
from jax.experimental.pallas import tpu_sc as plsc

_NUM_CORES = 2
_NUM_SUBCORES = 16
_NUM_WORKERS = _NUM_CORES * _NUM_SUBCORES


_HBM_SPLIT = 1536


def _remap_body(b_per_w, vocab, keys_hbm, table_hbm, out_hbm, idx_v, rows_v,
                tab_sh, sem_h, sem_s):
    s = lax.axis_index("s")
    wid = s * _NUM_CORES + lax.axis_index("c")
    base = wid * b_per_w
    n_s = b_per_w - _HBM_SPLIT
    pltpu.sync_copy(keys_hbm.at[pl.ds(base, b_per_w)], idx_v)
    hcopy = pltpu.async_copy(
        table_hbm.at[idx_v.at[pl.ds(0, _HBM_SPLIT)]],
        rows_v.at[pl.ds(0, _HBM_SPLIT)], sem_h)
    del vocab

    @pl.when(s == 0)
    def _stage_table():
        pltpu.sync_copy(table_hbm, tab_sh)

    plsc.subcore_barrier()
    scopy = pltpu.async_copy(
        tab_sh.at[idx_v.at[pl.ds(_HBM_SPLIT, n_s)]],
        rows_v.at[pl.ds(_HBM_SPLIT, n_s)], sem_s)
    hcopy.wait()
    scopy.wait()
    pltpu.sync_copy(rows_v, out_hbm.at[pl.ds(base, b_per_w)])


def kernel(offsets, keys, values, id2index):
    total = keys.shape[0]
    batch = offsets.shape[0]
    b_per_w = total // _NUM_WORKERS
    mesh = plsc.VectorSubcoreMesh(core_axis_name="c", subcore_axis_name="s")
    remap = pl.kernel(
        functools.partial(_remap_body, b_per_w, id2index.shape[0]),
        mesh=mesh,
        out_type=jax.ShapeDtypeStruct((total,), jnp.int32),
        scratch_types=[
            pltpu.VMEM((b_per_w,), jnp.int32),
            pltpu.VMEM((b_per_w,), jnp.int32),
            pltpu.VMEM_SHARED((id2index.shape[0],), jnp.int32),
            pltpu.SemaphoreType.DMA,
            pltpu.SemaphoreType.DMA,
        ],
    )
    idx_keys = remap(keys, id2index)

    def _tc_copy_body(vals_ref, offs_ref, vals_out, offs_out):
        vals_out[...] = vals_ref[...]
        offs_out[...] = offs_ref[...]

    vals2d = values.reshape(total // 128, 128)
    offs2d = offsets.reshape(batch // 128, 128)
    vals_out, offs_out = pl.pallas_call(
        _tc_copy_body,
        out_shape=(
            jax.ShapeDtypeStruct(vals2d.shape, values.dtype),
            jax.ShapeDtypeStruct(offs2d.shape, offsets.dtype),
        ),
    )(vals2d, offs2d)
    return (offs_out.reshape(batch), idx_keys, vals_out.reshape(total))
